# baseline (speedup 1.0000x reference)
import functools

import jax
import jax.numpy as jnp
from jax import lax
from jax.experimental import pallas as pl
from jax.experimental.pallas import tpu as pltpu
from jax.experimental.pallas import tpu_sc as plsc

_K = 16
_KROWS = 128
_ROWS = 256
_PROJ_ROWS = 1024
_GCHUNK = 128
_NC, _NS = 2, 16


def _proj_body(feat_ref, pts_ref, w1cT_ref, w1nT_ref, w1eT_ref, b1_ref,
               a_ref, g_ref):
    f = feat_ref[...]
    p = pts_ref[...]
    pc = jnp.dot(f, w1cT_ref[...], precision=lax.Precision.HIGHEST)
    pn = jnp.dot(f, w1nT_ref[...], precision=lax.Precision.HIGHEST)
    pe = jnp.dot(p, w1eT_ref[...], precision=lax.Precision.HIGHEST)
    a_ref[...] = pc - pe + b1_ref[...]
    g_ref[...] = pn + pe


def _proj_call(feat, pts_pad, w1cT, w1nT, w1eT, b1row):
    BN, C = feat.shape
    H = w1cT.shape[1]
    grid = (BN // _PROJ_ROWS,)
    return pl.pallas_call(
        _proj_body,
        grid=grid,
        in_specs=[
            pl.BlockSpec((_PROJ_ROWS, C), lambda i: (i, 0)),
            pl.BlockSpec((_PROJ_ROWS, 8), lambda i: (i, 0)),
            pl.BlockSpec((C, H), lambda i: (0, 0)),
            pl.BlockSpec((C, H), lambda i: (0, 0)),
            pl.BlockSpec((8, H), lambda i: (0, 0)),
            pl.BlockSpec((1, H), lambda i: (0, 0)),
        ],
        out_specs=[
            pl.BlockSpec((_PROJ_ROWS, H), lambda i: (i, 0)),
            pl.BlockSpec((_PROJ_ROWS, H), lambda i: (i, 0)),
        ],
        out_shape=[
            jax.ShapeDtypeStruct((BN, H), jnp.float32),
            jax.ShapeDtypeStruct((BN, H), jnp.float32),
        ],
    )(feat, pts_pad, w1cT, w1nT, w1eT, b1row)


_DEPTH = 5
_LANES = 64


def _knn_body(ptsS_ref, qT_ref, idx_ref):
    b = pl.program_id(0)
    xs = ptsS_ref[0]
    qT = qT_ref[0]
    dotT = jnp.dot(xs, qT, precision=lax.Precision.HIGHEST)
    sq_c = jnp.sum(xs * xs, axis=1, keepdims=True) * 0.25
    d2 = dotT + sq_c
    N, R = d2.shape
    ngrp = N // _LANES
    inf = jnp.float32(jnp.inf)

    sv = [jnp.full((_LANES, R), inf, jnp.float32) for _ in range(_DEPTH)]
    sg = [jnp.zeros((_LANES, R), jnp.int32) for _ in range(_DEPTH)]
    for v in range(ngrp):
        x = d2[v * _LANES:(v + 1) * _LANES, :]
        xid = jnp.full((_LANES, R), v, jnp.int32)
        for l in range(_DEPTH):
            pred = x < sv[l]
            ns = jnp.minimum(sv[l], x)
            nid = jnp.where(pred, xid, sg[l])
            if l + 1 < _DEPTH:
                nx = jnp.maximum(sv[l], x)
                nxid = jnp.where(pred, sg[l], xid)
                x, xid = nx, nxid
            sv[l], sg[l] = ns, nid

    pos = lax.broadcasted_iota(jnp.int32, (_LANES, R), 0)
    cm, cg = sv[0], sg[0]
    dep = jnp.zeros((_LANES, R), jnp.int32)
    ovf = jnp.zeros((_LANES, R), jnp.bool_)
    tails_v = sv[1:] + [jnp.full((_LANES, R), inf, jnp.float32)]
    tails_g = sg[1:] + [jnp.zeros((_LANES, R), jnp.int32)]
    cols = []
    for _ in range(_K):
        m = jnp.min(cm, axis=0, keepdims=True)
        pstar = jnp.min(jnp.where(cm <= m, pos, _LANES), axis=0,
                        keepdims=True)
        sel = pos == pstar
        ovf = ovf | (sel & (dep == _DEPTH - 1))
        gstar = jnp.min(jnp.where(sel, cg, ngrp), axis=0, keepdims=True)
        cols.append(gstar * _LANES + pstar)
        nv = tails_v[-1]
        ng = tails_g[-1]
        for t in range(len(tails_v) - 2, -1, -1):
            is_t = dep == t
            nv = jnp.where(is_t, tails_v[t], nv)
            ng = jnp.where(is_t, tails_g[t], ng)
        cm = jnp.where(sel, nv, cm)
        cg = jnp.where(sel, ng, cg)
        dep = jnp.where(sel, jnp.minimum(dep + 1, _DEPTH - 1), dep)
    idx_ref[0] = jnp.concatenate(cols, axis=0) + b * N

    @pl.when(jnp.any(ovf))
    def _slow():
        col = lax.broadcasted_iota(jnp.int32, (N, R), 0)
        vals = d2
        scols = []
        for _ in range(_K):
            mm = jnp.min(vals, axis=0, keepdims=True)
            i = jnp.min(jnp.where(vals <= mm, col, N), axis=0, keepdims=True)
            scols.append(i)
            vals = jnp.where(col == i, inf, vals)
        idx_ref[0] = jnp.concatenate(scols, axis=0) + b * N


def _knn_call(pts_scaled, ptsT):
    B, N, _ = pts_scaled.shape
    grid = (B, N // _KROWS)
    return pl.pallas_call(
        _knn_body,
        grid=grid,
        in_specs=[
            pl.BlockSpec((1, N, 8), lambda b, i: (b, 0, 0)),
            pl.BlockSpec((1, 8, _KROWS), lambda b, i: (b, 0, i)),
        ],
        out_specs=pl.BlockSpec((1, _K, _KROWS), lambda b, i: (b, 0, i)),
        out_shape=jax.ShapeDtypeStruct((B, _K, N), jnp.int32),
    )(pts_scaled, ptsT)


def _sc_gather_call(table, idx):
    M = idx.shape[0]
    D = table.shape[1]
    nw = _NC * _NS
    per_w = M // nw
    nchunk = per_w // _GCHUNK
    mesh = plsc.VectorSubcoreMesh(core_axis_name="c", subcore_axis_name="s")

    @functools.partial(
        pl.kernel, mesh=mesh,
        out_type=jax.ShapeDtypeStruct((M, D), jnp.float32),
        compiler_params=pltpu.CompilerParams(use_tc_tiling_on_sc=False),
        scratch_types=[
            pltpu.VMEM((_GCHUNK,), jnp.int32),
            pltpu.VMEM((_GCHUNK, D), jnp.float32),
            pltpu.SemaphoreType.DMA,
        ],
    )
    def gather_kernel(table_hbm, idx_hbm, out_hbm, idx_v, rows_v, sem):
        wid = lax.axis_index("s") * _NC + lax.axis_index("c")
        base = wid * per_w

        def body(c, carry):
            off = pl.multiple_of(base + c * _GCHUNK, _GCHUNK)
            pltpu.sync_copy(idx_hbm.at[pl.ds(off, _GCHUNK)], idx_v)
            pltpu.async_copy(table_hbm.at[idx_v], rows_v, sem).wait()
            pltpu.sync_copy(rows_v, out_hbm.at[pl.ds(off, _GCHUNK)])
            return carry

        lax.fori_loop(0, nchunk, body, 0)

    return gather_kernel(table, idx)


def _mlp_body(gath_ref, a_ref, w2T_ref, b2_ref, out_ref):
    g3 = gath_ref[0]
    a = a_ref[0]
    K, R, H = g3.shape
    h = jnp.maximum(g3 + a[None, :, :], 0.0)
    ef = jnp.dot(h.reshape(K * R, H), w2T_ref[...],
                 precision=lax.Precision.HIGHEST)
    ef = ef.reshape(K, R, ef.shape[-1])
    out_ref[0] = jnp.max(ef, axis=0) + b2_ref[...]


def _mlp_call(gath4, a3, w2T, b2row):
    B, K, N, H = gath4.shape
    CO = w2T.shape[1]
    grid = (B, N // _ROWS)
    return pl.pallas_call(
        _mlp_body,
        grid=grid,
        in_specs=[
            pl.BlockSpec((1, K, _ROWS, H), lambda b, i: (b, 0, i, 0)),
            pl.BlockSpec((1, _ROWS, H), lambda b, i: (b, i, 0)),
            pl.BlockSpec((H, CO), lambda b, i: (0, 0)),
            pl.BlockSpec((1, CO), lambda b, i: (0, 0)),
        ],
        out_specs=pl.BlockSpec((1, _ROWS, CO), lambda b, i: (b, i, 0)),
        out_shape=jax.ShapeDtypeStruct((B, N, CO), jnp.float32),
    )(gath4, a3, w2T, b2row)


def kernel(points, features, W1, b1, W2, b2):
    B, N, _ = points.shape
    C = features.shape[-1]
    H = W1.shape[0]
    CO = W2.shape[0]
    BN = B * N

    pts_pad = jnp.concatenate(
        [points, jnp.zeros((B, N, 5), points.dtype)], axis=-1)
    ptsT = jnp.swapaxes(pts_pad, 1, 2)
    pts_scaled = pts_pad * -2.0
    w1cT = jnp.transpose(W1[:, :C])
    w1nT = jnp.transpose(W1[:, C:2 * C])
    w1eT = jnp.transpose(jnp.concatenate(
        [W1[:, 2 * C:], jnp.zeros((H, 5), W1.dtype)], axis=1))
    w2T = jnp.transpose(W2)

    a, g = _proj_call(features.reshape(BN, C), pts_pad.reshape(BN, 8),
                      w1cT, w1nT, w1eT, b1.reshape(1, H))
    idx = _knn_call(pts_scaled, ptsT)
    gath = _sc_gather_call(g, idx.reshape(B * _K * N))
    out = _mlp_call(gath.reshape(B, _K, N, H), a.reshape(B, N, H),
                    w2T, b2.reshape(1, CO))
    return out


# speedup vs baseline: 1.0720x; 1.0720x over previous
import functools

import jax
import jax.numpy as jnp
from jax import lax
from jax.experimental import pallas as pl
from jax.experimental.pallas import tpu as pltpu
from jax.experimental.pallas import tpu_sc as plsc

_K = 16
_KROWS = 128
_ROWS = 256
_PROJ_ROWS = 1024
_GCHUNK = 128
_NC, _NS = 2, 16


def _proj_body(feat_ref, pts_ref, w1cT_ref, w1nT_ref, w1eT_ref, b1_ref,
               a_ref, g_ref):
    f = feat_ref[...]
    p = pts_ref[...]
    pc = jnp.dot(f, w1cT_ref[...], precision=lax.Precision.HIGHEST)
    pn = jnp.dot(f, w1nT_ref[...], precision=lax.Precision.HIGHEST)
    pe = jnp.dot(p, w1eT_ref[...], precision=lax.Precision.HIGHEST)
    a_ref[...] = pc - pe + b1_ref[...]
    g_ref[...] = pn + pe


def _proj_call(feat, pts_pad, w1cT, w1nT, w1eT, b1row):
    BN, C = feat.shape
    H = w1cT.shape[1]
    grid = (BN // _PROJ_ROWS,)
    return pl.pallas_call(
        _proj_body,
        grid=grid,
        in_specs=[
            pl.BlockSpec((_PROJ_ROWS, C), lambda i: (i, 0)),
            pl.BlockSpec((_PROJ_ROWS, 8), lambda i: (i, 0)),
            pl.BlockSpec((C, H), lambda i: (0, 0)),
            pl.BlockSpec((C, H), lambda i: (0, 0)),
            pl.BlockSpec((8, H), lambda i: (0, 0)),
            pl.BlockSpec((1, H), lambda i: (0, 0)),
        ],
        out_specs=[
            pl.BlockSpec((_PROJ_ROWS, H), lambda i: (i, 0)),
            pl.BlockSpec((_PROJ_ROWS, H), lambda i: (i, 0)),
        ],
        out_shape=[
            jax.ShapeDtypeStruct((BN, H), jnp.float32),
            jax.ShapeDtypeStruct((BN, H), jnp.float32),
        ],
    )(feat, pts_pad, w1cT, w1nT, w1eT, b1row)


_DEPTH = 5
_LANES = 64


def _knn_body(ptsS_ref, qT_ref, idx_ref):
    b = pl.program_id(0)
    xs = ptsS_ref[0]
    qT = qT_ref[0]
    dotT = jnp.dot(xs, qT, precision=lax.Precision.HIGHEST)
    sq_c = jnp.sum(xs * xs, axis=1, keepdims=True) * 0.25
    d2 = dotT + sq_c
    N, R = d2.shape
    ngrp = N // _LANES
    inf = jnp.float32(jnp.inf)

    sv = [jnp.full((_LANES, R), inf, jnp.float32) for _ in range(_DEPTH)]
    sg = [jnp.zeros((_LANES, R), jnp.int32) for _ in range(_DEPTH)]
    for v in range(ngrp):
        x = d2[v * _LANES:(v + 1) * _LANES, :]
        xid = jnp.full((_LANES, R), v, jnp.int32)
        for l in range(_DEPTH):
            pred = x < sv[l]
            ns = jnp.minimum(sv[l], x)
            nid = jnp.where(pred, xid, sg[l])
            if l + 1 < _DEPTH:
                nx = jnp.maximum(sv[l], x)
                nxid = jnp.where(pred, sg[l], xid)
                x, xid = nx, nxid
            sv[l], sg[l] = ns, nid

    pos = lax.broadcasted_iota(jnp.int32, (_LANES, R), 0)
    cm, cg = sv[0], sg[0]
    dep = jnp.zeros((_LANES, R), jnp.int32)
    ovf = jnp.zeros((_LANES, R), jnp.bool_)
    tails_v = sv[1:] + [jnp.full((_LANES, R), inf, jnp.float32)]
    tails_g = sg[1:] + [jnp.zeros((_LANES, R), jnp.int32)]
    cols = []
    for _ in range(_K):
        m = jnp.min(cm, axis=0, keepdims=True)
        pstar = jnp.min(jnp.where(cm <= m, pos, _LANES), axis=0,
                        keepdims=True)
        sel = pos == pstar
        ovf = ovf | (sel & (dep == _DEPTH - 1))
        gstar = jnp.min(jnp.where(sel, cg, ngrp), axis=0, keepdims=True)
        cols.append(gstar * _LANES + pstar)
        nv = tails_v[-1]
        ng = tails_g[-1]
        for t in range(len(tails_v) - 2, -1, -1):
            is_t = dep == t
            nv = jnp.where(is_t, tails_v[t], nv)
            ng = jnp.where(is_t, tails_g[t], ng)
        cm = jnp.where(sel, nv, cm)
        cg = jnp.where(sel, ng, cg)
        dep = jnp.where(sel, jnp.minimum(dep + 1, _DEPTH - 1), dep)
    idx_ref[0] = jnp.concatenate(cols, axis=0) + b * N

    @pl.when(jnp.any(ovf))
    def _slow():
        col = lax.broadcasted_iota(jnp.int32, (N, R), 0)
        vals = d2
        scols = []
        for _ in range(_K):
            mm = jnp.min(vals, axis=0, keepdims=True)
            i = jnp.min(jnp.where(vals <= mm, col, N), axis=0, keepdims=True)
            scols.append(i)
            vals = jnp.where(col == i, inf, vals)
        idx_ref[0] = jnp.concatenate(scols, axis=0) + b * N


def _knn_call(pts_scaled, ptsT):
    B, N, _ = pts_scaled.shape
    grid = (B, N // _KROWS)
    return pl.pallas_call(
        _knn_body,
        grid=grid,
        in_specs=[
            pl.BlockSpec((1, N, 8), lambda b, i: (b, 0, 0)),
            pl.BlockSpec((1, 8, _KROWS), lambda b, i: (b, 0, i)),
        ],
        out_specs=pl.BlockSpec((1, _K, _KROWS), lambda b, i: (b, 0, i)),
        out_shape=jax.ShapeDtypeStruct((B, _K, N), jnp.int32),
    )(pts_scaled, ptsT)


def _sc_gather_call(table, idx):
    M = idx.shape[0]
    D = table.shape[1]
    nw = _NC * _NS
    per_w = M // nw
    nchunk = per_w // _GCHUNK
    mesh = plsc.VectorSubcoreMesh(core_axis_name="c", subcore_axis_name="s")

    @functools.partial(
        pl.kernel, mesh=mesh,
        out_type=jax.ShapeDtypeStruct((M, D), jnp.float32),
        compiler_params=pltpu.CompilerParams(use_tc_tiling_on_sc=False),
        scratch_types=[
            pltpu.VMEM((_GCHUNK,), jnp.int32),
            pltpu.VMEM((_GCHUNK, D), jnp.float32),
            pltpu.SemaphoreType.DMA,
        ],
    )
    def gather_kernel(table_hbm, idx_hbm, out_hbm, idx_v, rows_v, sem):
        wid = lax.axis_index("s") * _NC + lax.axis_index("c")
        base = wid * per_w

        def body(c, carry):
            off = pl.multiple_of(base + c * _GCHUNK, _GCHUNK)
            pltpu.sync_copy(idx_hbm.at[pl.ds(off, _GCHUNK)], idx_v)
            pltpu.async_copy(table_hbm.at[idx_v], rows_v, sem).wait()
            pltpu.sync_copy(rows_v, out_hbm.at[pl.ds(off, _GCHUNK)])
            return carry

        lax.fori_loop(0, nchunk, body, 0)

    return gather_kernel(table, idx)


def _mlp_body(gath_ref, a_ref, w2T_ref, b2_ref, out_ref):
    g3 = gath_ref[0]
    a = a_ref[0]
    K, R, H = g3.shape
    h = jnp.maximum(g3 + a[None, :, :], 0.0)
    ef = jnp.dot(h.reshape(K * R, H), w2T_ref[...],
                 precision=lax.Precision.HIGHEST)
    ef = ef.reshape(K, R, ef.shape[-1])
    out_ref[0] = jnp.max(ef, axis=0) + b2_ref[...]


def _mlp_call(gath4, a3, w2T, b2row):
    B, K, N, H = gath4.shape
    CO = w2T.shape[1]
    grid = (B, N // _ROWS)
    return pl.pallas_call(
        _mlp_body,
        grid=grid,
        in_specs=[
            pl.BlockSpec((1, K, _ROWS, H), lambda b, i: (b, 0, i, 0)),
            pl.BlockSpec((1, _ROWS, H), lambda b, i: (b, i, 0)),
            pl.BlockSpec((H, CO), lambda b, i: (0, 0)),
            pl.BlockSpec((1, CO), lambda b, i: (0, 0)),
        ],
        out_specs=pl.BlockSpec((1, _ROWS, CO), lambda b, i: (b, i, 0)),
        out_shape=jax.ShapeDtypeStruct((B, N, CO), jnp.float32),
    )(gath4, a3, w2T, b2row)


def kernel(points, features, W1, b1, W2, b2):
    B, N, _ = points.shape
    C = features.shape[-1]
    H = W1.shape[0]
    CO = W2.shape[0]
    BN = B * N

    pts_pad = jnp.concatenate(
        [points, jnp.zeros((B, N, 5), points.dtype)], axis=-1)
    ptsT = jnp.swapaxes(pts_pad, 1, 2)
    pts_scaled = pts_pad * -2.0
    w1cT = jnp.transpose(W1[:, :C])
    w1nT = jnp.transpose(W1[:, C:2 * C])
    w1eT = jnp.transpose(jnp.concatenate(
        [W1[:, 2 * C:], jnp.zeros((H, 5), W1.dtype)], axis=1))
    w2T = jnp.transpose(W2)

    a, g = _proj_call(features.reshape(BN, C), pts_pad.reshape(BN, 8),
                      w1cT, w1nT, w1eT, b1.reshape(1, H))
    a3 = a.reshape(B, N, H)
    b2row = b2.reshape(1, CO)

    idxs = [_knn_call(pts_scaled[i:i + 1], ptsT[i:i + 1]) + i * N
            for i in range(B)]
    gaths = [_sc_gather_call(g, idxs[i].reshape(_K * N)) for i in range(B)]
    outs = [_mlp_call(gaths[i].reshape(1, _K, N, H), a3[i:i + 1], w2T, b2row)
            for i in range(B)]
    return jnp.concatenate(outs, axis=0)
